# Initial kernel scaffold; baseline (speedup 1.0000x reference)
import jax
import jax.numpy as jnp
from jax.experimental import pallas as pl


def kernel(x, pos, edge_index, batch, W1, b1, W2, b2, Wp, bp, Wz, bz):
    raise NotImplementedError("write your pallas kernel here")


# speedup vs baseline: 9.4589x; 9.4589x over previous
import functools

import jax
import jax.numpy as jnp
from jax import lax
from jax.experimental import pallas as pl
from jax.experimental.pallas import tpu as pltpu
from jax.experimental.pallas import tpu_sc as plsc

_N = 100000
_E = 3200000
_G = 64

_NC, _NS = 2, 16
_NW = _NC * _NS
_LANES = 128
_KR = 8
_CHUNK = _KR * _LANES
_NROWS = _E // _LANES
_NCHUNKS = _NROWS // _KR
_ZCH = 256
_NZ = 392
_NPAD = _NZ * _ZCH
_STRIPE = 6256

_BE = 12800
_BN = 4000


def _gather_body(t_hbm, src_hbm, dst_hbm, xs_hbm, xd_hbm,
                 idx_s, idx_d, buf_s, buf_d, sem):
    c = lax.axis_index("c")
    s = lax.axis_index("s")
    w = s * _NC + c

    def step(i, carry):
        k = w + _NW * i

        @pl.when(k < _NCHUNKS)
        def _():
            r0 = k * _KR
            e0 = r0 * _LANES
            pltpu.sync_copy(src_hbm.at[pl.ds(r0, _KR)], idx_s)
            pltpu.sync_copy(dst_hbm.at[pl.ds(r0, _KR)], idx_d)
            cps = [pltpu.async_copy(t_hbm.at[idx_s.at[j]],
                                    buf_s.at[pl.ds(j * _LANES, _LANES)], sem)
                   for j in range(_KR)]
            cpd = [pltpu.async_copy(t_hbm.at[idx_d.at[j]],
                                    buf_d.at[pl.ds(j * _LANES, _LANES)], sem)
                   for j in range(_KR)]
            for cp in cps + cpd:
                cp.wait()
            pltpu.sync_copy(buf_s, xs_hbm.at[pl.ds(e0, _CHUNK)])
            pltpu.sync_copy(buf_d, xd_hbm.at[pl.ds(e0, _CHUNK)])

        return carry

    lax.fori_loop(0, (_NCHUNKS + _NW - 1) // _NW, step, 0)


def _sc_mesh():
    return plsc.VectorSubcoreMesh(core_axis_name="c", subcore_axis_name="s",
                                  num_cores=_NC, num_subcores=_NS)


@functools.lru_cache(maxsize=None)
def _build_gather():
    return pl.kernel(
        _gather_body,
        out_type=(jax.ShapeDtypeStruct((_E, 8), jnp.float32),
                  jax.ShapeDtypeStruct((_E, 8), jnp.float32)),
        mesh=_sc_mesh(),
        scratch_types=[
            pltpu.VMEM((_KR, _LANES), jnp.int32),
            pltpu.VMEM((_KR, _LANES), jnp.int32),
            pltpu.VMEM((_CHUNK, 8), jnp.float32),
            pltpu.VMEM((_CHUNK, 8), jnp.float32),
            pltpu.SemaphoreType.DMA,
        ],
        compiler_params=pltpu.CompilerParams(use_tc_tiling_on_sc=False),
    )


def _gather_call(t, src2, dst2):
    return _build_gather()(t, src2, dst2)


def _mlp_body(xs_ref, xd_ref, w1_ref, b1_ref, ha_ref, hb_ref):
    xs = xs_ref[...]
    xd = xd_ref[...]
    d = xs[:, 4:7] - xd[:, 4:7]
    d2 = jnp.sum(d * d, axis=1, keepdims=True)
    dist = jnp.sqrt(d2)
    m = jnp.concatenate(
        [xd[:, 0:4], xs[:, 0:4], dist, jnp.zeros((_BE, 7), jnp.float32)],
        axis=1)
    pre = jnp.dot(m, w1_ref[...], preferred_element_type=jnp.float32)
    pre = pre + b1_ref[...]
    h = pre * jax.nn.sigmoid(pre)
    ha_ref[...] = h[:, 0:8]
    hb_ref[...] = h[:, 8:16]


def _mlp(xs, xd, w1p, b1r):
    return pl.pallas_call(
        _mlp_body,
        grid=(_E // _BE,),
        in_specs=[
            pl.BlockSpec((_BE, 8), lambda i: (i, 0)),
            pl.BlockSpec((_BE, 8), lambda i: (i, 0)),
            pl.BlockSpec((16, 16), lambda i: (0, 0)),
            pl.BlockSpec((1, 16), lambda i: (0, 0)),
        ],
        out_specs=[
            pl.BlockSpec((_BE, 8), lambda i: (i, 0)),
            pl.BlockSpec((_BE, 8), lambda i: (i, 0)),
        ],
        out_shape=[
            jax.ShapeDtypeStruct((_E, 8), jnp.float32),
            jax.ShapeDtypeStruct((_E, 8), jnp.float32),
        ],
    )(xs, xd, w1p, b1r)


def _make_scatter_body(with_count):
    def _scatter_body(*refs):
        if with_count:
            (h_hbm, dst_hbm, zrow_hbm, ones_hbm, psum_hbm, pcnt_hbm,
             idx_d, hbuf, obuf, zbuf, acc, accc) = refs
        else:
            (h_hbm, dst_hbm, zrow_hbm, psum_hbm,
             idx_d, hbuf, zbuf, acc) = refs

        c = lax.axis_index("c")
        s = lax.axis_index("s")

        pltpu.sync_copy(zrow_hbm, zbuf)
        if with_count:
            pltpu.sync_copy(ones_hbm, obuf)

        def zstep(i, carry):
            k = s + _NS * i

            @pl.when(k < _NZ)
            def _():
                pltpu.sync_copy(zbuf, acc.at[pl.ds(k * _ZCH, _ZCH)])
                if with_count:
                    pltpu.sync_copy(zbuf, accc.at[pl.ds(k * _ZCH, _ZCH)])

            return carry

        lax.fori_loop(0, (_NZ + _NS - 1) // _NS, zstep, 0)
        plsc.subcore_barrier()

        def step(i, carry):
            k = _NC * (s + _NS * i) + c

            @pl.when(k < _NCHUNKS)
            def _():
                r0 = k * _KR
                e0 = r0 * _LANES
                pltpu.sync_copy(dst_hbm.at[pl.ds(r0, _KR)], idx_d)
                pltpu.sync_copy(h_hbm.at[pl.ds(e0, _CHUNK)], hbuf)
                for j in range(_KR):
                    pltpu.sync_copy(hbuf.at[pl.ds(j * _LANES, _LANES)],
                                    acc.at[idx_d.at[j]], add=True)
                    if with_count:
                        pltpu.sync_copy(obuf, accc.at[idx_d.at[j]], add=True)

            return carry

        lax.fori_loop(0, (_NCHUNKS // _NC + _NS - 1) // _NS + 1, step, 0)
        plsc.subcore_barrier()

        def copy_out(stripe_len):
            base = s * _STRIPE
            pieces = [_CHUNK] * (stripe_len // _CHUNK)
            if stripe_len % _CHUNK:
                pieces.append(stripe_len % _CHUNK)
            off = 0
            for ln in pieces:
                r = base + off
                pltpu.sync_copy(acc.at[pl.ds(r, ln)], hbuf.at[pl.ds(0, ln)])
                pltpu.sync_copy(hbuf.at[pl.ds(0, ln)],
                                psum_hbm.at[c, pl.ds(r, ln)])
                if with_count:
                    pltpu.sync_copy(accc.at[pl.ds(r, ln)],
                                    hbuf.at[pl.ds(0, ln)])
                    pltpu.sync_copy(hbuf.at[pl.ds(0, ln)],
                                    pcnt_hbm.at[c, pl.ds(r, ln)])
                off += ln

        @pl.when(s < _NS - 1)
        def _():
            copy_out(_STRIPE)

        @pl.when(s == _NS - 1)
        def _():
            copy_out(_N - (_NS - 1) * _STRIPE)

    return _scatter_body


@functools.lru_cache(maxsize=None)
def _build_scatter(with_count):
    out_type = jax.ShapeDtypeStruct((_NC, _N, 8), jnp.float32)
    scratch = [
        pltpu.VMEM((_KR, _LANES), jnp.int32),
        pltpu.VMEM((_CHUNK, 8), jnp.float32),
    ]
    if with_count:
        out_type = (out_type, jax.ShapeDtypeStruct((_NC, _N, 8), jnp.float32))
        scratch.append(pltpu.VMEM((_LANES, 8), jnp.float32))
    scratch.append(pltpu.VMEM((_ZCH, 8), jnp.float32))
    scratch.append(pltpu.VMEM_SHARED((_NPAD, 8), jnp.float32))
    if with_count:
        scratch.append(pltpu.VMEM_SHARED((_NPAD, 8), jnp.float32))
    return pl.kernel(
        _make_scatter_body(with_count),
        out_type=out_type,
        mesh=_sc_mesh(),
        scratch_types=scratch,
        compiler_params=pltpu.CompilerParams(use_tc_tiling_on_sc=False),
    )


def _scatter_a(ha, dst2, zrow, ones):
    return _build_scatter(True)(ha, dst2, zrow, ones)


def _scatter_b(hb, dst2, zrow):
    return _build_scatter(False)(hb, dst2, zrow)


def _final_body(pa_ref, pc_ref, pb_ref, batch_ref, w2_ref, b2_ref,
                wp_ref, bp_ref, wz_ref, bz_ref, z_ref, s_ref, pooled):
    i = pl.program_id(0)
    pa = pa_ref[0] + pa_ref[1]
    pb = pb_ref[0] + pb_ref[1]
    pc = pc_ref[0] + pc_ref[1]
    ssum = jnp.concatenate([pa, pb], axis=1)
    cnt = pc[:, 0:1]
    agg = jnp.dot(ssum, w2_ref[...], preferred_element_type=jnp.float32)
    agg = (agg + cnt * b2_ref[...]) / jnp.maximum(cnt, 1.0)
    h = jnp.maximum(agg, 0.0)
    logits = jnp.dot(h, wp_ref[...], preferred_element_type=jnp.float32)
    logits = logits + bp_ref[...]
    mx = jnp.max(logits, axis=1, keepdims=True)
    ex = jnp.exp(logits - mx)
    sm = ex / jnp.sum(ex, axis=1, keepdims=True)
    s_ref[...] = sm
    w32 = jnp.concatenate([sm[:, 0:1] * h, sm[:, 1:2] * h], axis=1)
    b = batch_ref[...][:, 0]
    onehot = (lax.broadcasted_iota(jnp.int32, (_G, _BN), 0)
              == b[None, :]).astype(jnp.float32)
    contrib = jnp.dot(onehot, w32, preferred_element_type=jnp.float32)

    @pl.when(i == 0)
    def _():
        pooled[...] = contrib

    @pl.when(i > 0)
    def _():
        pooled[...] += contrib

    @pl.when(i == _N // _BN - 1)
    def _():
        z_ref[...] = (jnp.dot(pooled[...], wz_ref[...],
                              preferred_element_type=jnp.float32)
                      + bz_ref[...])


def _final(pa, pc, pb, batch2, w2, b2r, wp, bpr, wz, bzr):
    return pl.pallas_call(
        _final_body,
        grid=(_N // _BN,),
        in_specs=[
            pl.BlockSpec((_NC, _BN, 8), lambda i: (0, i, 0)),
            pl.BlockSpec((_NC, _BN, 8), lambda i: (0, i, 0)),
            pl.BlockSpec((_NC, _BN, 8), lambda i: (0, i, 0)),
            pl.BlockSpec((_BN, 1), lambda i: (i, 0)),
            pl.BlockSpec((16, 16), lambda i: (0, 0)),
            pl.BlockSpec((1, 16), lambda i: (0, 0)),
            pl.BlockSpec((16, 2), lambda i: (0, 0)),
            pl.BlockSpec((1, 2), lambda i: (0, 0)),
            pl.BlockSpec((32, 8), lambda i: (0, 0)),
            pl.BlockSpec((1, 8), lambda i: (0, 0)),
        ],
        out_specs=[
            pl.BlockSpec((_G, 8), lambda i: (0, 0)),
            pl.BlockSpec((_BN, 2), lambda i: (i, 0)),
        ],
        out_shape=[
            jax.ShapeDtypeStruct((_G, 8), jnp.float32),
            jax.ShapeDtypeStruct((_N, 2), jnp.float32),
        ],
        scratch_shapes=[pltpu.VMEM((_G, 32), jnp.float32)],
    )(pa, pc, pb, batch2, w2, b2r, wp, bpr, wz, bzr)


def kernel(x, pos, edge_index, batch, W1, b1, W2, b2, Wp, bp, Wz, bz):
    src = edge_index[0].reshape(_NROWS, _LANES)
    dst = edge_index[1].reshape(_NROWS, _LANES)
    t = jnp.concatenate([x, pos, jnp.zeros((_N, 1), jnp.float32)], axis=1)
    xs, xd = _gather_call(t, src, dst)
    w1p = jnp.pad(W1, ((0, 7), (0, 0)))
    ha, hb = _mlp(xs, xd, w1p, b1.reshape(1, 16))
    zrow = jnp.zeros((_ZCH, 8), jnp.float32)
    ones = jnp.ones((_LANES, 8), jnp.float32)
    pa, pc = _scatter_a(ha, dst, zrow, ones)
    pb = _scatter_b(hb, dst, zrow)
    z, s = _final(pa, pc, pb, batch.reshape(_N, 1), W2, b2.reshape(1, 16),
                  Wp, bp.reshape(1, 2), Wz, bz.reshape(1, 8))
    return z, s


# speedup vs baseline: 40.3301x; 4.2637x over previous
import functools

import jax
import jax.numpy as jnp
from jax import lax
from jax.experimental import pallas as pl
from jax.experimental.pallas import tpu as pltpu
from jax.experimental.pallas import tpu_sc as plsc

_N = 100000
_E = 3200000
_G = 64

_NC, _NS = 2, 16
_NW = _NC * _NS
_LANES = 128
_KR = 8
_CHUNK = _KR * _LANES
_NROWS = _E // _LANES
_NCHUNKS = _NROWS // _KR
_ZCH = 256
_NZ = 392
_NPAD = _NZ * _ZCH
_STRIPE = 6256

_BE = 12800
_BN = 4000


def _gather_body(t_hbm, src_hbm, dst_hbm, xs_hbm, xd_hbm,
                 idx_s, idx_d, buf_s, buf_d, sem):
    c = lax.axis_index("c")
    s = lax.axis_index("s")
    w = s * _NC + c

    def step(i, carry):
        k = w + _NW * i

        @pl.when(k < _NCHUNKS)
        def _():
            r0 = k * _KR
            e0 = r0 * _LANES
            pltpu.sync_copy(src_hbm.at[pl.ds(r0, _KR)], idx_s)
            pltpu.sync_copy(dst_hbm.at[pl.ds(r0, _KR)], idx_d)
            cps = [pltpu.async_copy(t_hbm.at[idx_s.at[j]],
                                    buf_s.at[pl.ds(j * _LANES, _LANES)], sem)
                   for j in range(_KR)]
            cpd = [pltpu.async_copy(t_hbm.at[idx_d.at[j]],
                                    buf_d.at[pl.ds(j * _LANES, _LANES)], sem)
                   for j in range(_KR)]
            for cp in cps + cpd:
                cp.wait()
            pltpu.sync_copy(buf_s, xs_hbm.at[pl.ds(e0, _CHUNK)])
            pltpu.sync_copy(buf_d, xd_hbm.at[pl.ds(e0, _CHUNK)])

        return carry

    lax.fori_loop(0, (_NCHUNKS + _NW - 1) // _NW, step, 0)


def _sc_mesh():
    return plsc.VectorSubcoreMesh(core_axis_name="c", subcore_axis_name="s",
                                  num_cores=_NC, num_subcores=_NS)


@functools.lru_cache(maxsize=None)
def _build_gather():
    return pl.kernel(
        _gather_body,
        out_type=(jax.ShapeDtypeStruct((_E, 8), jnp.float32),
                  jax.ShapeDtypeStruct((_E, 8), jnp.float32)),
        mesh=_sc_mesh(),
        scratch_types=[
            pltpu.VMEM((_KR, _LANES), jnp.int32),
            pltpu.VMEM((_KR, _LANES), jnp.int32),
            pltpu.VMEM((_CHUNK, 8), jnp.float32),
            pltpu.VMEM((_CHUNK, 8), jnp.float32),
            pltpu.SemaphoreType.DMA,
        ],
        compiler_params=pltpu.CompilerParams(use_tc_tiling_on_sc=False),
    )


def _gather_call(t, src2, dst2):
    return _build_gather()(t, src2, dst2)


_BEP = _BE // 16
_EP = _E // 16


def _mlp_body(xs_ref, xd_ref, a_ref, b_ref, c_ref, bias_ref, sel_ref,
              ha_ref, hb_ref):
    xs = xs_ref[...]
    xd = xd_ref[...]
    u = xs - xd
    d2 = jnp.dot(u * u, sel_ref[...], preferred_element_type=jnp.float32)
    dist = jnp.sqrt(d2)
    pre = (jnp.dot(xd, a_ref[...], preferred_element_type=jnp.float32)
           + jnp.dot(xs, b_ref[...], preferred_element_type=jnp.float32)
           + jnp.dot(dist, c_ref[...], preferred_element_type=jnp.float32)
           + bias_ref[...])
    h = pre * jax.nn.sigmoid(pre)
    ha_ref[...] = h[:, 0:128]
    hb_ref[...] = h[:, 128:256]


def _mlp(xsp, xdp, a_bd, b_bd, c_bd, bias_bd, sel_bd):
    return pl.pallas_call(
        _mlp_body,
        grid=(_EP // _BEP,),
        in_specs=[
            pl.BlockSpec((_BEP, 128), lambda i: (i, 0)),
            pl.BlockSpec((_BEP, 128), lambda i: (i, 0)),
            pl.BlockSpec((128, 256), lambda i: (0, 0)),
            pl.BlockSpec((128, 256), lambda i: (0, 0)),
            pl.BlockSpec((16, 256), lambda i: (0, 0)),
            pl.BlockSpec((1, 256), lambda i: (0, 0)),
            pl.BlockSpec((128, 16), lambda i: (0, 0)),
        ],
        out_specs=[
            pl.BlockSpec((_BEP, 128), lambda i: (i, 0)),
            pl.BlockSpec((_BEP, 128), lambda i: (i, 0)),
        ],
        out_shape=[
            jax.ShapeDtypeStruct((_EP, 128), jnp.float32),
            jax.ShapeDtypeStruct((_EP, 128), jnp.float32),
        ],
    )(xsp, xdp, a_bd, b_bd, c_bd, bias_bd, sel_bd)


def _mlp_weights(W1, b1):
    eye = jnp.eye(16, dtype=jnp.float32)
    wd = jnp.concatenate([W1[0:4], jnp.zeros((4, 16), jnp.float32)], axis=0)
    ws = jnp.concatenate([W1[4:8], jnp.zeros((4, 16), jnp.float32)], axis=0)
    a_bd = jnp.concatenate([jnp.kron(eye, wd[:, 0:8]),
                            jnp.kron(eye, wd[:, 8:16])], axis=1)
    b_bd = jnp.concatenate([jnp.kron(eye, ws[:, 0:8]),
                            jnp.kron(eye, ws[:, 8:16])], axis=1)
    c_bd = jnp.concatenate([jnp.kron(eye, W1[8:9, 0:8]),
                            jnp.kron(eye, W1[8:9, 8:16])], axis=1)
    bias_bd = jnp.concatenate([jnp.tile(b1[0:8], 16),
                               jnp.tile(b1[8:16], 16)]).reshape(1, 256)
    sel8 = jnp.array([0, 0, 0, 0, 1, 1, 1, 0],
                     jnp.float32).reshape(8, 1)
    sel_bd = jnp.kron(eye, sel8)
    return a_bd, b_bd, c_bd, bias_bd, sel_bd


def _make_scatter_body(with_count):
    def _scatter_body(*refs):
        if with_count:
            (h_hbm, dst_hbm, zrow_hbm, ones_hbm, psum_hbm, pcnt_hbm,
             idx_d, hbuf, obuf, zbuf, acc, accc) = refs
        else:
            (h_hbm, dst_hbm, zrow_hbm, psum_hbm,
             idx_d, hbuf, zbuf, acc) = refs

        c = lax.axis_index("c")
        s = lax.axis_index("s")

        pltpu.sync_copy(zrow_hbm, zbuf)
        if with_count:
            pltpu.sync_copy(ones_hbm, obuf)

        def zstep(i, carry):
            k = s + _NS * i

            @pl.when(k < _NZ)
            def _():
                pltpu.sync_copy(zbuf, acc.at[pl.ds(k * _ZCH, _ZCH)])
                if with_count:
                    pltpu.sync_copy(zbuf, accc.at[pl.ds(k * _ZCH, _ZCH)])

            return carry

        lax.fori_loop(0, (_NZ + _NS - 1) // _NS, zstep, 0)
        plsc.subcore_barrier()

        def step(i, carry):
            k = _NC * (s + _NS * i) + c

            @pl.when(k < _NCHUNKS)
            def _():
                r0 = k * _KR
                e0 = r0 * _LANES
                pltpu.sync_copy(dst_hbm.at[pl.ds(r0, _KR)], idx_d)
                pltpu.sync_copy(h_hbm.at[pl.ds(e0, _CHUNK)], hbuf)
                for j in range(_KR):
                    pltpu.sync_copy(hbuf.at[pl.ds(j * _LANES, _LANES)],
                                    acc.at[idx_d.at[j]], add=True)
                    if with_count:
                        pltpu.sync_copy(obuf, accc.at[idx_d.at[j]], add=True)

            return carry

        lax.fori_loop(0, (_NCHUNKS // _NC + _NS - 1) // _NS + 1, step, 0)
        plsc.subcore_barrier()

        def copy_out(stripe_len):
            base = s * _STRIPE
            pieces = [_CHUNK] * (stripe_len // _CHUNK)
            if stripe_len % _CHUNK:
                pieces.append(stripe_len % _CHUNK)
            off = 0
            for ln in pieces:
                r = base + off
                pltpu.sync_copy(acc.at[pl.ds(r, ln)], hbuf.at[pl.ds(0, ln)])
                pltpu.sync_copy(hbuf.at[pl.ds(0, ln)],
                                psum_hbm.at[c, pl.ds(r, ln)])
                if with_count:
                    pltpu.sync_copy(accc.at[pl.ds(r, ln)],
                                    hbuf.at[pl.ds(0, ln)])
                    pltpu.sync_copy(hbuf.at[pl.ds(0, ln)],
                                    pcnt_hbm.at[c, pl.ds(r, ln)])
                off += ln

        @pl.when(s < _NS - 1)
        def _():
            copy_out(_STRIPE)

        @pl.when(s == _NS - 1)
        def _():
            copy_out(_N - (_NS - 1) * _STRIPE)

    return _scatter_body


@functools.lru_cache(maxsize=None)
def _build_scatter(with_count):
    out_type = jax.ShapeDtypeStruct((_NC, _N, 8), jnp.float32)
    scratch = [
        pltpu.VMEM((_KR, _LANES), jnp.int32),
        pltpu.VMEM((_CHUNK, 8), jnp.float32),
    ]
    if with_count:
        out_type = (out_type, jax.ShapeDtypeStruct((_NC, _N, 8), jnp.float32))
        scratch.append(pltpu.VMEM((_LANES, 8), jnp.float32))
    scratch.append(pltpu.VMEM((_ZCH, 8), jnp.float32))
    scratch.append(pltpu.VMEM_SHARED((_NPAD, 8), jnp.float32))
    if with_count:
        scratch.append(pltpu.VMEM_SHARED((_NPAD, 8), jnp.float32))
    return pl.kernel(
        _make_scatter_body(with_count),
        out_type=out_type,
        mesh=_sc_mesh(),
        scratch_types=scratch,
        compiler_params=pltpu.CompilerParams(use_tc_tiling_on_sc=False),
    )


def _scatter_a(ha, dst2, zrow, ones):
    return _build_scatter(True)(ha, dst2, zrow, ones)


def _scatter_b(hb, dst2, zrow):
    return _build_scatter(False)(hb, dst2, zrow)


def _final_body(pa_ref, pc_ref, pb_ref, batch_ref, w2_ref, b2_ref,
                wp_ref, bp_ref, wz_ref, bz_ref, z_ref, s_ref, pooled):
    i = pl.program_id(0)
    pa = pa_ref[0] + pa_ref[1]
    pb = pb_ref[0] + pb_ref[1]
    pc = pc_ref[0] + pc_ref[1]
    ssum = jnp.concatenate([pa, pb], axis=1)
    cnt = pc[:, 0:1]
    agg = jnp.dot(ssum, w2_ref[...], preferred_element_type=jnp.float32)
    agg = (agg + cnt * b2_ref[...]) / jnp.maximum(cnt, 1.0)
    h = jnp.maximum(agg, 0.0)
    logits = jnp.dot(h, wp_ref[...], preferred_element_type=jnp.float32)
    logits = logits + bp_ref[...]
    mx = jnp.max(logits, axis=1, keepdims=True)
    ex = jnp.exp(logits - mx)
    sm = ex / jnp.sum(ex, axis=1, keepdims=True)
    s_ref[...] = sm
    w32 = jnp.concatenate([sm[:, 0:1] * h, sm[:, 1:2] * h], axis=1)
    b = batch_ref[...][:, 0]
    onehot = (lax.broadcasted_iota(jnp.int32, (_G, _BN), 0)
              == b[None, :]).astype(jnp.float32)
    contrib = jnp.dot(onehot, w32, preferred_element_type=jnp.float32)

    @pl.when(i == 0)
    def _():
        pooled[...] = contrib

    @pl.when(i > 0)
    def _():
        pooled[...] += contrib

    @pl.when(i == _N // _BN - 1)
    def _():
        z_ref[...] = (jnp.dot(pooled[...], wz_ref[...],
                              preferred_element_type=jnp.float32)
                      + bz_ref[...])


def _final(pa, pc, pb, batch2, w2, b2r, wp, bpr, wz, bzr):
    return pl.pallas_call(
        _final_body,
        grid=(_N // _BN,),
        in_specs=[
            pl.BlockSpec((_NC, _BN, 8), lambda i: (0, i, 0)),
            pl.BlockSpec((_NC, _BN, 8), lambda i: (0, i, 0)),
            pl.BlockSpec((_NC, _BN, 8), lambda i: (0, i, 0)),
            pl.BlockSpec((_BN, 1), lambda i: (i, 0)),
            pl.BlockSpec((16, 16), lambda i: (0, 0)),
            pl.BlockSpec((1, 16), lambda i: (0, 0)),
            pl.BlockSpec((16, 2), lambda i: (0, 0)),
            pl.BlockSpec((1, 2), lambda i: (0, 0)),
            pl.BlockSpec((32, 8), lambda i: (0, 0)),
            pl.BlockSpec((1, 8), lambda i: (0, 0)),
        ],
        out_specs=[
            pl.BlockSpec((_G, 8), lambda i: (0, 0)),
            pl.BlockSpec((_BN, 2), lambda i: (i, 0)),
        ],
        out_shape=[
            jax.ShapeDtypeStruct((_G, 8), jnp.float32),
            jax.ShapeDtypeStruct((_N, 2), jnp.float32),
        ],
        scratch_shapes=[pltpu.VMEM((_G, 32), jnp.float32)],
    )(pa, pc, pb, batch2, w2, b2r, wp, bpr, wz, bzr)


def kernel(x, pos, edge_index, batch, W1, b1, W2, b2, Wp, bp, Wz, bz):
    src = edge_index[0].reshape(_NROWS, _LANES)
    dst = edge_index[1].reshape(_NROWS, _LANES)
    t = jnp.concatenate([x, pos, jnp.zeros((_N, 1), jnp.float32)], axis=1)
    xs, xd = _gather_call(t, src, dst)
    a_bd, b_bd, c_bd, bias_bd, sel_bd = _mlp_weights(W1, b1)
    hap, hbp = _mlp(xs.reshape(_EP, 128), xd.reshape(_EP, 128),
                    a_bd, b_bd, c_bd, bias_bd, sel_bd)
    ha = hap.reshape(_E, 8)
    hb = hbp.reshape(_E, 8)
    zrow = jnp.zeros((_ZCH, 8), jnp.float32)
    ones = jnp.ones((_LANES, 8), jnp.float32)
    pa, pc = _scatter_a(ha, dst, zrow, ones)
    pb = _scatter_b(hb, dst, zrow)
    z, s = _final(pa, pc, pb, batch.reshape(_N, 1), W2, b2.reshape(1, 16),
                  Wp, bp.reshape(1, 2), Wz, bz.reshape(1, 8))
    return z, s
